# Initial kernel scaffold; baseline (speedup 1.0000x reference)
import jax
import jax.numpy as jnp
from jax.experimental import pallas as pl


def kernel(cons_x, var_x, edge_index, edge_attr, break_indicator, cons_shift, cons_scale, cons_W1, cons_b1, cons_W2, cons_b2, var_shift, var_scale, var_W1, var_b1, var_W2, var_b2, edge_shift, edge_scale, break_W, lin_l_W, lin_l_b, lin_r_W):
    raise NotImplementedError("write your pallas kernel here")


# speedup vs baseline: 3.2667x; 3.2667x over previous
import functools

import jax
import jax.numpy as jnp
from jax import lax
from jax.experimental import pallas as pl
from jax.experimental.pallas import tpu as pltpu
from jax.experimental.pallas import tpu_sc as plsc

N_NODE = 50000
N_EDGE = 800000
EMB = 64
HALF = 32

NC = 2
NS = 16
B = 128
BPT = 400
CH = 50
NCH = BPT // CH
EP = NS * BPT * B
ACC_R = 50048
STRIPE = ACC_R // NS


def _fill(buf, nrows, width, value):
    vec = jnp.full((16,), value, jnp.float32)

    def fv(i, carry):
        for j in range(width // 16):
            buf[i, pl.ds(j * 16, 16)] = vec
        return carry

    lax.fori_loop(0, nrows, fv, 0)


def _mega_body(tab1, tab2, g1, g2, s1, s2, cidx, out_v, out_c, out_n,
               g_i, s_i, rows0, rows1, acc, sem0, sem1):
    c = lax.axis_index("c")
    s = lax.axis_index("s")

    def zero_acc():
        _fill(rows0, B, HALF, 0.0)

        def zs(k, carry):
            pltpu.sync_copy(rows0, acc.at[pl.ds(s * STRIPE + k * B, B)])
            return carry

        lax.fori_loop(0, STRIPE // B, zs, 0)
        rem = STRIPE - (STRIPE // B) * B
        pltpu.sync_copy(rows0.at[pl.ds(0, rem)],
                        acc.at[pl.ds(s * STRIPE + (STRIPE // B) * B, rem)])

    def relation(table, gidx, sidx, out):
        zero_acc()
        plsc.subcore_barrier()

        def chunk(k, carry):
            pltpu.sync_copy(gidx.at[c, pl.ds(s * BPT + k * CH, CH)], g_i)
            pltpu.sync_copy(sidx.at[pl.ds(s * BPT + k * CH, CH)], s_i)

            def pair(p, carry2):
                b0 = 2 * p
                b1 = b0 + 1
                cp0 = pltpu.async_copy(table.at[g_i.at[b0]], rows0, sem0)
                cp1 = pltpu.async_copy(table.at[g_i.at[b1]], rows1, sem1)
                cp0.wait()
                pltpu.sync_copy(rows0, acc.at[s_i.at[b0]], add=True)
                cp1.wait()
                pltpu.sync_copy(rows1, acc.at[s_i.at[b1]], add=True)
                return carry2

            lax.fori_loop(0, CH // 2, pair, 0)
            return carry

        lax.fori_loop(0, NCH, chunk, 0)
        plsc.subcore_barrier()
        pltpu.sync_copy(acc.at[pl.ds(s * STRIPE, STRIPE)],
                        out.at[c, pl.ds(s * STRIPE, STRIPE)])

    relation(tab1, g1, s1, out_v)
    relation(tab2, g2, s2, out_c)

    zero_acc()
    _fill(rows0, B, HALF, 1.0)
    plsc.subcore_barrier()

    def cchunk(k, carry):
        pltpu.sync_copy(cidx.at[c, pl.ds(s * BPT + k * CH, CH)], s_i)

        def bat(b, carry2):
            pltpu.sync_copy(rows0, acc.at[s_i.at[b]], add=True)
            return carry2

        lax.fori_loop(0, CH, bat, 0)
        return carry

    lax.fori_loop(0, NCH, cchunk, 0)
    plsc.subcore_barrier()
    pltpu.sync_copy(acc.at[pl.ds(s * STRIPE, STRIPE)],
                    out_n.at[c, pl.ds(s * STRIPE, STRIPE)])


@functools.cache
def _get_mega():
    mesh = plsc.VectorSubcoreMesh(core_axis_name="c", subcore_axis_name="s",
                                  num_cores=NC, num_subcores=NS)
    acc_ty = jax.ShapeDtypeStruct((NC, ACC_R, HALF), jnp.float32)
    return functools.partial(
        pl.kernel,
        out_type=[acc_ty, acc_ty, acc_ty],
        mesh=mesh,
        scratch_types=[
            pltpu.VMEM((CH, B), jnp.int32),
            pltpu.VMEM((CH, B), jnp.int32),
            pltpu.VMEM((B, HALF), jnp.float32),
            pltpu.VMEM((B, HALF), jnp.float32),
            pltpu.VMEM_SHARED((ACC_R, HALF), jnp.float32),
            pltpu.SemaphoreType.DMA,
            pltpu.SemaphoreType.DMA,
        ],
        compiler_params=pltpu.CompilerParams(use_tc_tiling_on_sc=False,
                                             has_side_effects=True),
    )(_mega_body)


def _mega(*args):
    return _get_mega()(*args)


RB = 1000
GRID = N_NODE // RB

def _bcast(i):
    return (0, 0)


def _embed_body(cx, vx, bi, cw1, cb1, cw2, cb2, vw1, vb1, vw2, vb2, bw,
                ch_o, vh_o):
    f32 = jnp.float32
    ch = jnp.maximum(jnp.dot(cx[...], cw1[...], preferred_element_type=f32)
                     + cb1[...], 0.0)
    ch = jnp.maximum(jnp.dot(ch, cw2[...], preferred_element_type=f32)
                     + cb2[...], 0.0)
    vh = jnp.maximum(jnp.dot(vx[...], vw1[...], preferred_element_type=f32)
                     + vb1[...], 0.0)
    vh = jnp.maximum(jnp.dot(vh, vw2[...], preferred_element_type=f32)
                     + vb2[...], 0.0)
    vh = vh + bi[...] * bw[...]
    ch_o[...] = ch
    vh_o[...] = vh


def _make_embed():
    wspec = lambda shp: pl.BlockSpec(shp, _bcast)
    return pl.pallas_call(
        _embed_body,
        grid=(GRID,),
        in_specs=[
            pl.BlockSpec((RB, 8), lambda i: (i, 0)),
            pl.BlockSpec((RB, 24), lambda i: (i, 0)),
            pl.BlockSpec((RB, 1), lambda i: (i, 0)),
            wspec((8, EMB)), wspec((1, EMB)),
            wspec((EMB, EMB)), wspec((1, EMB)),
            wspec((24, EMB)), wspec((1, EMB)),
            wspec((EMB, EMB)), wspec((1, EMB)),
            wspec((1, EMB)),
        ],
        out_specs=[
            pl.BlockSpec((RB, EMB), lambda i: (i, 0)),
            pl.BlockSpec((RB, EMB), lambda i: (i, 0)),
        ],
        out_shape=[
            jax.ShapeDtypeStruct((N_NODE, EMB), jnp.float32),
            jax.ShapeDtypeStruct((N_NODE, EMB), jnp.float32),
        ],
    )


def _upd_body(sv, sc_, cnts, vh0, ch0, wl0, bl0, wr0, wl1, bl1, wr1,
              vh_o, ch_o):
    f32 = jnp.float32
    sv_a = sv[...]
    sc_a = sc_[...]
    cn_a = cnts[...]
    mean_v = (jnp.concatenate([sv_a[0], sv_a[1]], axis=1)
              / jnp.maximum(cn_a[0][:, 0:1], 1.0))
    nv = (jnp.dot(mean_v, wl0[...], preferred_element_type=f32) + bl0[...]
          + jnp.dot(vh0[...], wr0[...], preferred_element_type=f32))
    mean_c = (jnp.concatenate([sc_a[0], sc_a[1]], axis=1)
              / jnp.maximum(cn_a[1][:, 0:1], 1.0))
    ncn = (jnp.dot(mean_c, wl1[...], preferred_element_type=f32) + bl1[...]
           + jnp.dot(ch0[...], wr1[...], preferred_element_type=f32))
    vh_o[...] = jnp.maximum(nv, 0.0)
    ch_o[...] = jnp.maximum(ncn, 0.0)


def _make_upd():
    wspec = lambda shp: pl.BlockSpec(shp, _bcast)
    sspec = pl.BlockSpec((NC, RB, HALF), lambda i: (0, i, 0))
    return pl.pallas_call(
        _upd_body,
        grid=(GRID,),
        in_specs=[
            sspec, sspec, sspec,
            pl.BlockSpec((RB, EMB), lambda i: (i, 0)),
            pl.BlockSpec((RB, EMB), lambda i: (i, 0)),
            wspec((EMB, EMB)), wspec((1, EMB)), wspec((EMB, EMB)),
            wspec((EMB, EMB)), wspec((1, EMB)), wspec((EMB, EMB)),
        ],
        out_specs=[
            pl.BlockSpec((RB, EMB), lambda i: (i, 0)),
            pl.BlockSpec((RB, EMB), lambda i: (i, 0)),
        ],
        out_shape=[
            jax.ShapeDtypeStruct((N_NODE, EMB), jnp.float32),
            jax.ShapeDtypeStruct((N_NODE, EMB), jnp.float32),
        ],
    )


_embed_call = _make_embed()
_upd_call = _make_upd()


def kernel(cons_x, var_x, edge_index, edge_attr, break_indicator,
           cons_shift, cons_scale, cons_W1, cons_b1, cons_W2, cons_b2,
           var_shift, var_scale, var_W1, var_b1, var_W2, var_b2,
           edge_shift, edge_scale, break_W, lin_l_W, lin_l_b, lin_r_W):
    del edge_attr, edge_shift, edge_scale

    cw1 = cons_scale[:, None] * cons_W1
    cb1 = (cons_b1 + (cons_shift * cons_scale) @ cons_W1)[None, :]
    vw1 = var_scale[:, None] * var_W1
    vb1 = (var_b1 + (var_shift * var_scale) @ var_W1)[None, :]
    cx = jnp.pad(cons_x, ((0, 0), (0, 3)))
    vx = jnp.pad(var_x, ((0, 0), (0, 5)))
    cw1 = jnp.pad(cw1, ((0, 3), (0, 0)))
    vw1 = jnp.pad(vw1, ((0, 5), (0, 0)))

    src = edge_index[0].astype(jnp.int32)
    dst = edge_index[1].astype(jnp.int32)
    padn = EP - N_EDGE
    src_g = jnp.pad(src, (0, padn))
    dst_g = jnp.pad(dst, (0, padn))
    src_s = jnp.pad(src, (0, padn), constant_values=N_NODE)
    dst_s = jnp.pad(dst, (0, padn), constant_values=N_NODE)

    def gidx_of(x):
        return jnp.stack([2 * x, 2 * x + 1]).reshape(NC, NS * BPT, B)

    g_rel1 = gidx_of(src_g)
    g_rel2 = gidx_of(dst_g)
    s_rel1 = dst_s.reshape(NS * BPT, B)
    s_rel2 = src_s.reshape(NS * BPT, B)
    c_idx = jnp.stack([dst_s, src_s]).reshape(NC, NS * BPT, B)

    ch0, vh0 = _embed_call(cx, vx, break_indicator, cw1, cb1,
                           cons_W2, cons_b2[None, :], vw1, vb1,
                           var_W2, var_b2[None, :], break_W)

    nlayers = lax.optimization_barrier(jnp.int32(2))

    def cond(st):
        return st[0] < nlayers

    def body(st):
        i, vh, ch = st
        take = lambda a: lax.dynamic_index_in_dim(a, i, 0, keepdims=False)
        wl = take(lin_l_W)
        bl = take(lin_l_b)
        wr = take(lin_r_W)
        sv, sc_, cn = _mega(ch.reshape(NC * N_NODE, HALF),
                            vh.reshape(NC * N_NODE, HALF),
                            g_rel1, g_rel2, s_rel1, s_rel2, c_idx)
        vh_n, ch_n = _upd_call(sv, sc_, cn, vh, ch,
                               wl[0], bl[0][None, :], wr[0],
                               wl[1], bl[1][None, :], wr[1])
        return (i + 1, vh_n, ch_n)

    _, vh2, _ = lax.while_loop(cond, body, (jnp.int32(0), vh0, ch0))
    return vh2


# speedup vs baseline: 3.6432x; 1.1152x over previous
import functools

import jax
import jax.numpy as jnp
from jax import lax
from jax.experimental import pallas as pl
from jax.experimental.pallas import tpu as pltpu
from jax.experimental.pallas import tpu_sc as plsc

N_NODE = 50000
N_EDGE = 800000
EMB = 64
HALF = 32

NC = 2
NS = 16
B = 128
BPT = 400
CH = 16
NCH = BPT // CH
NB = 4
EP = NS * BPT * B
ACC_R = 50048
STRIPE = ACC_R // NS


def _fill(buf, nrows, width, value):
    vec = jnp.full((16,), value, jnp.float32)

    def fv(i, carry):
        for j in range(width // 16):
            buf[i, pl.ds(j * 16, 16)] = vec
        return carry

    lax.fori_loop(0, nrows, fv, 0)


def _mega_body(tab1, tab2, g1, g2, s1, s2, cidx, out_v, out_c, out_n,
               g_i, s_i, r0, r1, r2, r3, acc,
               gs0, gs1, gs2, gs3, ss0, ss1, ss2, ss3, ig, isx):
    c = lax.axis_index("c")
    s = lax.axis_index("s")
    rows = [r0, r1, r2, r3]
    gsem = [gs0, gs1, gs2, gs3]
    ssem = [ss0, ss1, ss2, ss3]

    def zero_acc():
        _fill(r0, B, HALF, 0.0)

        def zs(k, carry):
            pltpu.sync_copy(r0, acc.at[pl.ds(s * STRIPE + k * B, B)])
            return carry

        lax.fori_loop(0, STRIPE // B, zs, 0)
        rem = STRIPE - (STRIPE // B) * B
        pltpu.sync_copy(r0.at[pl.ds(0, rem)],
                        acc.at[pl.ds(s * STRIPE + (STRIPE // B) * B, rem)])

    def relation(table, gidx, sidx, out):
        zero_acc()
        plsc.subcore_barrier()
        pltpu.async_copy(gidx.at[c, pl.ds(s * BPT, CH)], g_i.at[0], ig)
        pltpu.async_copy(sidx.at[pl.ds(s * BPT, CH)], s_i.at[0], isx)

        def chunk(k, carry):
            cur = lax.rem(k, 2)
            nxt = 1 - cur
            pltpu.make_async_copy(gidx.at[c, pl.ds(0, CH)],
                                  g_i.at[cur], ig).wait()
            pltpu.make_async_copy(sidx.at[pl.ds(0, CH)],
                                  s_i.at[cur], isx).wait()

            @pl.when(k + 1 < NCH)
            def _():
                off = s * BPT + (k + 1) * CH
                pltpu.async_copy(gidx.at[c, pl.ds(off, CH)], g_i.at[nxt], ig)
                pltpu.async_copy(sidx.at[pl.ds(off, CH)], s_i.at[nxt], isx)

            gd = [None] * CH
            sd = [None] * CH
            for p in range(CH):
                slot = p % NB
                if p >= NB:
                    sd[p - NB].wait()
                gd[p] = pltpu.async_copy(table.at[g_i.at[cur, p]],
                                         rows[slot], gsem[slot])
                if p >= 2:
                    q = p - 2
                    gd[q].wait()
                    sd[q] = pltpu.async_copy(rows[q % NB],
                                             acc.at[s_i.at[cur, q]],
                                             ssem[q % NB], add=True)
            for q in range(CH - 2, CH):
                gd[q].wait()
                sd[q] = pltpu.async_copy(rows[q % NB],
                                         acc.at[s_i.at[cur, q]],
                                         ssem[q % NB], add=True)
            for q in range(CH - NB, CH):
                sd[q].wait()
            return carry

        lax.fori_loop(0, NCH, chunk, 0)
        plsc.subcore_barrier()
        pltpu.sync_copy(acc.at[pl.ds(s * STRIPE, STRIPE)],
                        out.at[c, pl.ds(s * STRIPE, STRIPE)])

    relation(tab1, g1, s1, out_v)
    relation(tab2, g2, s2, out_c)

    zero_acc()
    _fill(r0, B, HALF, 1.0)
    plsc.subcore_barrier()
    pltpu.async_copy(cidx.at[c, pl.ds(s * BPT, CH)], s_i.at[0], isx)

    def cchunk(k, carry):
        cur = lax.rem(k, 2)
        nxt = 1 - cur
        pltpu.make_async_copy(cidx.at[c, pl.ds(0, CH)],
                              s_i.at[cur], isx).wait()

        @pl.when(k + 1 < NCH)
        def _():
            off = s * BPT + (k + 1) * CH
            pltpu.async_copy(cidx.at[c, pl.ds(off, CH)], s_i.at[nxt], isx)

        sd = [None] * CH
        for p in range(CH):
            if p >= NB:
                sd[p - NB].wait()
            sd[p] = pltpu.async_copy(r0, acc.at[s_i.at[cur, p]],
                                     ssem[p % NB], add=True)
        for q in range(CH - NB, CH):
            sd[q].wait()
        return carry

    lax.fori_loop(0, NCH, cchunk, 0)
    plsc.subcore_barrier()
    pltpu.sync_copy(acc.at[pl.ds(s * STRIPE, STRIPE)],
                    out_n.at[c, pl.ds(s * STRIPE, STRIPE)])


@functools.cache
def _get_mega():
    mesh = plsc.VectorSubcoreMesh(core_axis_name="c", subcore_axis_name="s",
                                  num_cores=NC, num_subcores=NS)
    acc_ty = jax.ShapeDtypeStruct((NC, ACC_R, HALF), jnp.float32)
    return functools.partial(
        pl.kernel,
        out_type=[acc_ty, acc_ty, acc_ty],
        mesh=mesh,
        scratch_types=[
            pltpu.VMEM((2, CH, B), jnp.int32),
            pltpu.VMEM((2, CH, B), jnp.int32),
            pltpu.VMEM((B, HALF), jnp.float32),
            pltpu.VMEM((B, HALF), jnp.float32),
            pltpu.VMEM((B, HALF), jnp.float32),
            pltpu.VMEM((B, HALF), jnp.float32),
            pltpu.VMEM_SHARED((ACC_R, HALF), jnp.float32),
            pltpu.SemaphoreType.DMA,
            pltpu.SemaphoreType.DMA,
            pltpu.SemaphoreType.DMA,
            pltpu.SemaphoreType.DMA,
            pltpu.SemaphoreType.DMA,
            pltpu.SemaphoreType.DMA,
            pltpu.SemaphoreType.DMA,
            pltpu.SemaphoreType.DMA,
            pltpu.SemaphoreType.DMA,
            pltpu.SemaphoreType.DMA,
        ],
        compiler_params=pltpu.CompilerParams(use_tc_tiling_on_sc=False,
                                             has_side_effects=True),
    )(_mega_body)


def _mega(*args):
    return _get_mega()(*args)


RB = 1000
GRID = N_NODE // RB

def _bcast(i):
    return (0, 0)


def _embed_body(cx, vx, bi, cw1, cb1, cw2, cb2, vw1, vb1, vw2, vb2, bw,
                ch_o, vh_o):
    f32 = jnp.float32
    ch = jnp.maximum(jnp.dot(cx[...], cw1[...], preferred_element_type=f32)
                     + cb1[...], 0.0)
    ch = jnp.maximum(jnp.dot(ch, cw2[...], preferred_element_type=f32)
                     + cb2[...], 0.0)
    vh = jnp.maximum(jnp.dot(vx[...], vw1[...], preferred_element_type=f32)
                     + vb1[...], 0.0)
    vh = jnp.maximum(jnp.dot(vh, vw2[...], preferred_element_type=f32)
                     + vb2[...], 0.0)
    vh = vh + bi[...] * bw[...]
    ch_o[...] = ch
    vh_o[...] = vh


def _make_embed():
    wspec = lambda shp: pl.BlockSpec(shp, _bcast)
    return pl.pallas_call(
        _embed_body,
        grid=(GRID,),
        in_specs=[
            pl.BlockSpec((RB, 8), lambda i: (i, 0)),
            pl.BlockSpec((RB, 24), lambda i: (i, 0)),
            pl.BlockSpec((RB, 1), lambda i: (i, 0)),
            wspec((8, EMB)), wspec((1, EMB)),
            wspec((EMB, EMB)), wspec((1, EMB)),
            wspec((24, EMB)), wspec((1, EMB)),
            wspec((EMB, EMB)), wspec((1, EMB)),
            wspec((1, EMB)),
        ],
        out_specs=[
            pl.BlockSpec((RB, EMB), lambda i: (i, 0)),
            pl.BlockSpec((RB, EMB), lambda i: (i, 0)),
        ],
        out_shape=[
            jax.ShapeDtypeStruct((N_NODE, EMB), jnp.float32),
            jax.ShapeDtypeStruct((N_NODE, EMB), jnp.float32),
        ],
    )


def _upd_body(sv, sc_, cnts, vh0, ch0, wl0, bl0, wr0, wl1, bl1, wr1,
              vh_o, ch_o):
    f32 = jnp.float32
    sv_a = sv[...]
    sc_a = sc_[...]
    cn_a = cnts[...]
    mean_v = (jnp.concatenate([sv_a[0], sv_a[1]], axis=1)
              / jnp.maximum(cn_a[0][:, 0:1], 1.0))
    nv = (jnp.dot(mean_v, wl0[...], preferred_element_type=f32) + bl0[...]
          + jnp.dot(vh0[...], wr0[...], preferred_element_type=f32))
    mean_c = (jnp.concatenate([sc_a[0], sc_a[1]], axis=1)
              / jnp.maximum(cn_a[1][:, 0:1], 1.0))
    ncn = (jnp.dot(mean_c, wl1[...], preferred_element_type=f32) + bl1[...]
           + jnp.dot(ch0[...], wr1[...], preferred_element_type=f32))
    vh_o[...] = jnp.maximum(nv, 0.0)
    ch_o[...] = jnp.maximum(ncn, 0.0)


def _make_upd():
    wspec = lambda shp: pl.BlockSpec(shp, _bcast)
    sspec = pl.BlockSpec((NC, RB, HALF), lambda i: (0, i, 0))
    return pl.pallas_call(
        _upd_body,
        grid=(GRID,),
        in_specs=[
            sspec, sspec, sspec,
            pl.BlockSpec((RB, EMB), lambda i: (i, 0)),
            pl.BlockSpec((RB, EMB), lambda i: (i, 0)),
            wspec((EMB, EMB)), wspec((1, EMB)), wspec((EMB, EMB)),
            wspec((EMB, EMB)), wspec((1, EMB)), wspec((EMB, EMB)),
        ],
        out_specs=[
            pl.BlockSpec((RB, EMB), lambda i: (i, 0)),
            pl.BlockSpec((RB, EMB), lambda i: (i, 0)),
        ],
        out_shape=[
            jax.ShapeDtypeStruct((N_NODE, EMB), jnp.float32),
            jax.ShapeDtypeStruct((N_NODE, EMB), jnp.float32),
        ],
    )


_embed_call = _make_embed()
_upd_call = _make_upd()


def kernel(cons_x, var_x, edge_index, edge_attr, break_indicator,
           cons_shift, cons_scale, cons_W1, cons_b1, cons_W2, cons_b2,
           var_shift, var_scale, var_W1, var_b1, var_W2, var_b2,
           edge_shift, edge_scale, break_W, lin_l_W, lin_l_b, lin_r_W):
    del edge_attr, edge_shift, edge_scale

    cw1 = cons_scale[:, None] * cons_W1
    cb1 = (cons_b1 + (cons_shift * cons_scale) @ cons_W1)[None, :]
    vw1 = var_scale[:, None] * var_W1
    vb1 = (var_b1 + (var_shift * var_scale) @ var_W1)[None, :]
    cx = jnp.pad(cons_x, ((0, 0), (0, 3)))
    vx = jnp.pad(var_x, ((0, 0), (0, 5)))
    cw1 = jnp.pad(cw1, ((0, 3), (0, 0)))
    vw1 = jnp.pad(vw1, ((0, 5), (0, 0)))

    src = edge_index[0].astype(jnp.int32)
    dst = edge_index[1].astype(jnp.int32)
    padn = EP - N_EDGE
    src_g = jnp.pad(src, (0, padn))
    dst_g = jnp.pad(dst, (0, padn))
    src_s = jnp.pad(src, (0, padn), constant_values=N_NODE)
    dst_s = jnp.pad(dst, (0, padn), constant_values=N_NODE)

    def gidx_of(x):
        return jnp.stack([2 * x, 2 * x + 1]).reshape(NC, NS * BPT, B)

    g_rel1 = gidx_of(src_g)
    g_rel2 = gidx_of(dst_g)
    s_rel1 = dst_s.reshape(NS * BPT, B)
    s_rel2 = src_s.reshape(NS * BPT, B)
    c_idx = jnp.stack([dst_s, src_s]).reshape(NC, NS * BPT, B)

    ch0, vh0 = _embed_call(cx, vx, break_indicator, cw1, cb1,
                           cons_W2, cons_b2[None, :], vw1, vb1,
                           var_W2, var_b2[None, :], break_W)

    nlayers = lax.optimization_barrier(jnp.int32(2))

    def cond(st):
        return st[0] < nlayers

    def body(st):
        i, vh, ch = st
        take = lambda a: lax.dynamic_index_in_dim(a, i, 0, keepdims=False)
        wl = take(lin_l_W)
        bl = take(lin_l_b)
        wr = take(lin_r_W)
        sv, sc_, cn = _mega(ch.reshape(NC * N_NODE, HALF),
                            vh.reshape(NC * N_NODE, HALF),
                            g_rel1, g_rel2, s_rel1, s_rel2, c_idx)
        vh_n, ch_n = _upd_call(sv, sc_, cn, vh, ch,
                               wl[0], bl[0][None, :], wr[0],
                               wl[1], bl[1][None, :], wr[1])
        return (i + 1, vh_n, ch_n)

    _, vh2, _ = lax.while_loop(cond, body, (jnp.int32(0), vh0, ch0))
    return vh2


# speedup vs baseline: 4.3186x; 1.1854x over previous
import functools

import jax
import jax.numpy as jnp
from jax import lax
from jax.experimental import pallas as pl
from jax.experimental.pallas import tpu as pltpu
from jax.experimental.pallas import tpu_sc as plsc

N_NODE = 50000
N_EDGE = 800000
EMB = 64
QW = 16
NQ = EMB // QW

NC = 2
NS = 16
B = 128
BPT = 400
NB = 4
EP = NS * BPT * B
ACC_R = 50048
STRIPE = ACC_R // NS


def _fill(buf, nrows, width, value):
    vec = jnp.full((16,), value, jnp.float32)

    def fv(i, carry):
        for j in range(width // 16):
            buf[i, pl.ds(j * 16, 16)] = vec
        return carry

    lax.fori_loop(0, nrows, fv, 0)


def _zero_acc(acc, buf, s):
    _fill(buf, B, QW, 0.0)

    def zs(k, carry):
        pltpu.sync_copy(buf, acc.at[pl.ds(s * STRIPE + k * B, B)])
        return carry

    lax.fori_loop(0, STRIPE // B, zs, 0)
    rem = STRIPE - (STRIPE // B) * B
    pltpu.sync_copy(buf.at[pl.ds(0, rem)],
                    acc.at[pl.ds(s * STRIPE + (STRIPE // B) * B, rem)])


def _relation_round(table, gidx, q, sidx, out, ch, c, s,
                    g_i, s_i, rows, gsem, ssem, ig, isx, acc):
    nch = BPT // ch
    _zero_acc(acc, rows[0], s)
    plsc.subcore_barrier()
    pltpu.async_copy(gidx.at[q, pl.ds(s * BPT, ch)], g_i.at[0], ig)
    pltpu.async_copy(sidx.at[pl.ds(s * BPT, ch)], s_i.at[0], isx)

    def chunk(k, carry):
        cur = lax.rem(k, 2)
        nxt = 1 - cur
        pltpu.make_async_copy(gidx.at[q, pl.ds(0, ch)],
                              g_i.at[cur], ig).wait()
        pltpu.make_async_copy(sidx.at[pl.ds(0, ch)], s_i.at[cur], isx).wait()

        @pl.when(k + 1 < nch)
        def _():
            off = s * BPT + (k + 1) * ch
            pltpu.async_copy(gidx.at[q, pl.ds(off, ch)], g_i.at[nxt], ig)
            pltpu.async_copy(sidx.at[pl.ds(off, ch)], s_i.at[nxt], isx)

        gd = [None] * ch
        sd = [None] * ch

        def scat(p):
            gd[p].wait()
            sd[p] = pltpu.async_copy(rows[p % NB], acc.at[s_i.at[cur, p]],
                                     ssem[p % NB], add=True)

        for p in range(ch):
            if p >= NB:
                sd[p - NB].wait()
            gd[p] = pltpu.async_copy(table.at[g_i.at[cur, p]],
                                     rows[p % NB], gsem[p % NB])
            if p >= 2:
                scat(p - 2)
        for p in range(ch - 2, ch):
            scat(p)
        for p in range(ch - NB, ch):
            sd[p].wait()
        return carry

    lax.fori_loop(0, nch, chunk, 0)
    plsc.subcore_barrier()
    pltpu.sync_copy(acc.at[pl.ds(s * STRIPE, STRIPE)],
                    out.at[q, pl.ds(s * STRIPE, STRIPE)])


def _hist(cidx, out, ch, c, s, s_i, ones, ssem, isx, acc):
    nch = BPT // ch
    _zero_acc(acc, ones, s)
    _fill(ones, B, QW, 1.0)
    plsc.subcore_barrier()
    pltpu.async_copy(cidx.at[c, pl.ds(s * BPT, ch)], s_i.at[0], isx)

    def chunk(k, carry):
        cur = lax.rem(k, 2)
        nxt = 1 - cur
        pltpu.make_async_copy(cidx.at[c, pl.ds(0, ch)],
                              s_i.at[cur], isx).wait()

        @pl.when(k + 1 < nch)
        def _():
            off = s * BPT + (k + 1) * ch
            pltpu.async_copy(cidx.at[c, pl.ds(off, ch)], s_i.at[nxt], isx)

        sd = [None] * ch
        for p in range(ch):
            if p >= NB:
                sd[p - NB].wait()
            sd[p] = pltpu.async_copy(ones, acc.at[s_i.at[cur, p]],
                                     ssem[p % NB], add=True)
        for p in range(ch - NB, ch):
            sd[p].wait()
        return carry

    lax.fori_loop(0, nch, chunk, 0)
    plsc.subcore_barrier()
    pltpu.sync_copy(acc.at[pl.ds(s * STRIPE, STRIPE)],
                    out.at[c, pl.ds(s * STRIPE, STRIPE)])


CH_A = 16
CH_B = 8


def _layer1_body(tab_c, tab_v, g1, g2, s1, s2, cidx, out_v, out_c, out_n,
                 g_i, s_i, r0, r1, r2, r3,
                 gs0, gs1, gs2, gs3, ss0, ss1, ss2, ss3, ig, isx, acc):
    c = lax.axis_index("c")
    s = lax.axis_index("s")
    rows = [r0, r1, r2, r3]
    gsem = [gs0, gs1, gs2, gs3]
    ssem = [ss0, ss1, ss2, ss3]
    for r in range(2):
        _relation_round(tab_c, g1, 2 * c + r, s1, out_v, CH_A, c, s,
                        g_i, s_i, rows, gsem, ssem, ig, isx, acc)
    for r in range(2):
        _relation_round(tab_v, g2, 2 * c + r, s2, out_c, CH_A, c, s,
                        g_i, s_i, rows, gsem, ssem, ig, isx, acc)
    _hist(cidx, out_n, CH_A, c, s, s_i, r0, ssem, isx, acc)


def _layer2_body(tab_c, g1, s1, out_v,
                 g_i, s_i, r0, r1, r2, r3,
                 gs0, gs1, gs2, gs3, ss0, ss1, ss2, ss3, ig, isx, acc):
    c = lax.axis_index("c")
    s = lax.axis_index("s")
    rows = [r0, r1, r2, r3]
    gsem = [gs0, gs1, gs2, gs3]
    ssem = [ss0, ss1, ss2, ss3]
    for r in range(2):
        _relation_round(tab_c, g1, 2 * c + r, s1, out_v, CH_B, c, s,
                        g_i, s_i, rows, gsem, ssem, ig, isx, acc)


def _sc_scratch(ch):
    return [
        pltpu.VMEM((2, ch, B), jnp.int32),
        pltpu.VMEM((2, ch, B), jnp.int32),
        pltpu.VMEM((B, QW), jnp.float32),
        pltpu.VMEM((B, QW), jnp.float32),
        pltpu.VMEM((B, QW), jnp.float32),
        pltpu.VMEM((B, QW), jnp.float32),
    ] + [pltpu.SemaphoreType.DMA] * 10 + [
        pltpu.VMEM_SHARED((ACC_R, QW), jnp.float32),
    ]


@functools.cache
def _get_layer1():
    mesh = plsc.VectorSubcoreMesh(core_axis_name="c", subcore_axis_name="s",
                                  num_cores=NC, num_subcores=NS)
    sum_ty = jax.ShapeDtypeStruct((NQ, ACC_R, QW), jnp.float32)
    cnt_ty = jax.ShapeDtypeStruct((NC, ACC_R, QW), jnp.float32)
    return functools.partial(
        pl.kernel,
        out_type=[sum_ty, sum_ty, cnt_ty],
        mesh=mesh,
        scratch_types=_sc_scratch(CH_A),
        compiler_params=pltpu.CompilerParams(use_tc_tiling_on_sc=False),
    )(_layer1_body)


@functools.cache
def _get_layer2():
    mesh = plsc.VectorSubcoreMesh(core_axis_name="c", subcore_axis_name="s",
                                  num_cores=NC, num_subcores=NS)
    sum_ty = jax.ShapeDtypeStruct((NQ, ACC_R, QW), jnp.float32)
    return functools.partial(
        pl.kernel,
        out_type=sum_ty,
        mesh=mesh,
        scratch_types=_sc_scratch(CH_B),
        compiler_params=pltpu.CompilerParams(use_tc_tiling_on_sc=False),
    )(_layer2_body)


def _sc_layer1(*args):
    return _get_layer1()(*args)


def _sc_layer2(*args):
    return _get_layer2()(*args)


RB = 1000
GRID = N_NODE // RB

def _bcast(i):
    return (0, 0)


def _embed_body(cx, vx, bi, cw1, cb1, cw2, cb2, vw1, vb1, vw2, vb2, bw,
                ch_o, vh_o):
    f32 = jnp.float32
    ch = jnp.maximum(jnp.dot(cx[...], cw1[...], preferred_element_type=f32)
                     + cb1[...], 0.0)
    ch = jnp.maximum(jnp.dot(ch, cw2[...], preferred_element_type=f32)
                     + cb2[...], 0.0)
    vh = jnp.maximum(jnp.dot(vx[...], vw1[...], preferred_element_type=f32)
                     + vb1[...], 0.0)
    vh = jnp.maximum(jnp.dot(vh, vw2[...], preferred_element_type=f32)
                     + vb2[...], 0.0)
    vh = vh + bi[...] * bw[...]
    ch_o[...] = ch
    vh_o[...] = vh


def _make_embed():
    wspec = lambda shp: pl.BlockSpec(shp, _bcast)
    return pl.pallas_call(
        _embed_body,
        grid=(GRID,),
        in_specs=[
            pl.BlockSpec((RB, 8), lambda i: (i, 0)),
            pl.BlockSpec((RB, 24), lambda i: (i, 0)),
            pl.BlockSpec((RB, 1), lambda i: (i, 0)),
            wspec((8, EMB)), wspec((1, EMB)),
            wspec((EMB, EMB)), wspec((1, EMB)),
            wspec((24, EMB)), wspec((1, EMB)),
            wspec((EMB, EMB)), wspec((1, EMB)),
            wspec((1, EMB)),
        ],
        out_specs=[
            pl.BlockSpec((RB, EMB), lambda i: (i, 0)),
            pl.BlockSpec((RB, EMB), lambda i: (i, 0)),
        ],
        out_shape=[
            jax.ShapeDtypeStruct((N_NODE, EMB), jnp.float32),
            jax.ShapeDtypeStruct((N_NODE, EMB), jnp.float32),
        ],
    )


def _upd1_body(sv, sc_, cnts, vh0, ch0, wl0, bl0, wr0, wl1, bl1, wr1,
               vh_o, ch_o):
    f32 = jnp.float32
    sv_a = sv[...]
    sc_a = sc_[...]
    cn_a = cnts[...]
    mean_v = (jnp.concatenate([sv_a[q] for q in range(NQ)], axis=1)
              / jnp.maximum(cn_a[0][:, 0:1], 1.0))
    nv = (jnp.dot(mean_v, wl0[...], preferred_element_type=f32) + bl0[...]
          + jnp.dot(vh0[...], wr0[...], preferred_element_type=f32))
    mean_c = (jnp.concatenate([sc_a[q] for q in range(NQ)], axis=1)
              / jnp.maximum(cn_a[1][:, 0:1], 1.0))
    ncn = (jnp.dot(mean_c, wl1[...], preferred_element_type=f32) + bl1[...]
           + jnp.dot(ch0[...], wr1[...], preferred_element_type=f32))
    vh_o[...] = jnp.maximum(nv, 0.0)
    ch_o[...] = jnp.maximum(ncn, 0.0)


def _make_upd1():
    wspec = lambda shp: pl.BlockSpec(shp, _bcast)
    sspec = pl.BlockSpec((NQ, RB, QW), lambda i: (0, i, 0))
    return pl.pallas_call(
        _upd1_body,
        grid=(GRID,),
        in_specs=[
            sspec, sspec,
            pl.BlockSpec((NC, RB, QW), lambda i: (0, i, 0)),
            pl.BlockSpec((RB, EMB), lambda i: (i, 0)),
            pl.BlockSpec((RB, EMB), lambda i: (i, 0)),
            wspec((EMB, EMB)), wspec((1, EMB)), wspec((EMB, EMB)),
            wspec((EMB, EMB)), wspec((1, EMB)), wspec((EMB, EMB)),
        ],
        out_specs=[
            pl.BlockSpec((RB, EMB), lambda i: (i, 0)),
            pl.BlockSpec((RB, EMB), lambda i: (i, 0)),
        ],
        out_shape=[
            jax.ShapeDtypeStruct((N_NODE, EMB), jnp.float32),
            jax.ShapeDtypeStruct((N_NODE, EMB), jnp.float32),
        ],
    )


def _upd2_body(sv, cnts, vh1, wl, bl, wr, vh_o):
    f32 = jnp.float32
    sv_a = sv[...]
    cn_a = cnts[...]
    mean_v = (jnp.concatenate([sv_a[q] for q in range(NQ)], axis=1)
              / jnp.maximum(cn_a[0][:, 0:1], 1.0))
    nv = (jnp.dot(mean_v, wl[...], preferred_element_type=f32) + bl[...]
          + jnp.dot(vh1[...], wr[...], preferred_element_type=f32))
    vh_o[...] = jnp.maximum(nv, 0.0)


def _make_upd2():
    wspec = lambda shp: pl.BlockSpec(shp, _bcast)
    return pl.pallas_call(
        _upd2_body,
        grid=(GRID,),
        in_specs=[
            pl.BlockSpec((NQ, RB, QW), lambda i: (0, i, 0)),
            pl.BlockSpec((NC, RB, QW), lambda i: (0, i, 0)),
            pl.BlockSpec((RB, EMB), lambda i: (i, 0)),
            wspec((EMB, EMB)), wspec((1, EMB)), wspec((EMB, EMB)),
        ],
        out_specs=pl.BlockSpec((RB, EMB), lambda i: (i, 0)),
        out_shape=jax.ShapeDtypeStruct((N_NODE, EMB), jnp.float32),
    )


_embed_call = _make_embed()
_upd1_call = _make_upd1()
_upd2_call = _make_upd2()


def kernel(cons_x, var_x, edge_index, edge_attr, break_indicator,
           cons_shift, cons_scale, cons_W1, cons_b1, cons_W2, cons_b2,
           var_shift, var_scale, var_W1, var_b1, var_W2, var_b2,
           edge_shift, edge_scale, break_W, lin_l_W, lin_l_b, lin_r_W):
    del edge_attr, edge_shift, edge_scale

    cw1 = cons_scale[:, None] * cons_W1
    cb1 = (cons_b1 + (cons_shift * cons_scale) @ cons_W1)[None, :]
    vw1 = var_scale[:, None] * var_W1
    vb1 = (var_b1 + (var_shift * var_scale) @ var_W1)[None, :]
    cx = jnp.pad(cons_x, ((0, 0), (0, 3)))
    vx = jnp.pad(var_x, ((0, 0), (0, 5)))
    cw1 = jnp.pad(cw1, ((0, 3), (0, 0)))
    vw1 = jnp.pad(vw1, ((0, 5), (0, 0)))

    src = edge_index[0].astype(jnp.int32)
    dst = edge_index[1].astype(jnp.int32)
    padn = EP - N_EDGE
    src_g = jnp.pad(src, (0, padn))
    dst_g = jnp.pad(dst, (0, padn))
    src_s = jnp.pad(src, (0, padn), constant_values=N_NODE)
    dst_s = jnp.pad(dst, (0, padn), constant_values=N_NODE)

    def gidx_of(x):
        return jnp.stack([NQ * x + q for q in range(NQ)]).reshape(
            NQ, NS * BPT, B)

    g_rel1 = gidx_of(src_g)
    g_rel2 = gidx_of(dst_g)
    s_rel1 = dst_s.reshape(NS * BPT, B)
    s_rel2 = src_s.reshape(NS * BPT, B)
    c_idx = jnp.stack([dst_s, src_s]).reshape(NC, NS * BPT, B)

    ch0, vh0 = _embed_call(cx, vx, break_indicator, cw1, cb1,
                           cons_W2, cons_b2[None, :], vw1, vb1,
                           var_W2, var_b2[None, :], break_W)

    sv1, sc1, cn = _sc_layer1(ch0.reshape(NQ * N_NODE, QW),
                              vh0.reshape(NQ * N_NODE, QW),
                              g_rel1, g_rel2, s_rel1, s_rel2, c_idx)
    vh1, ch1 = _upd1_call(sv1, sc1, cn, vh0, ch0,
                          lin_l_W[0, 0], lin_l_b[0, 0][None, :],
                          lin_r_W[0, 0],
                          lin_l_W[0, 1], lin_l_b[0, 1][None, :],
                          lin_r_W[0, 1])

    sv2 = _sc_layer2(ch1.reshape(NQ * N_NODE, QW), g_rel1, s_rel1)
    vh2 = _upd2_call(sv2, cn, vh1,
                     lin_l_W[1, 0], lin_l_b[1, 0][None, :], lin_r_W[1, 0])
    return vh2


# speedup vs baseline: 4.5115x; 1.0447x over previous
import functools

import jax
import jax.numpy as jnp
from jax import lax
from jax.experimental import pallas as pl
from jax.experimental.pallas import tpu as pltpu
from jax.experimental.pallas import tpu_sc as plsc

N_NODE = 50000
N_EDGE = 800000
EMB = 64
QW = 16
NQ = EMB // QW

NC = 2
NS = 16
B = 128
BPT = 400
NB = 4
EP = NS * BPT * B
ACC_R = 50048
STRIPE = ACC_R // NS


def _fill(buf, nrows, width, value):
    vec = jnp.full((16,), value, jnp.float32)

    def fv(i, carry):
        for j in range(width // 16):
            buf[i, pl.ds(j * 16, 16)] = vec
        return carry

    lax.fori_loop(0, nrows, fv, 0)


def _zero_acc(acc, buf, s):
    _fill(buf, B, QW, 0.0)

    def zs(k, carry):
        pltpu.sync_copy(buf, acc.at[pl.ds(s * STRIPE + k * B, B)])
        return carry

    lax.fori_loop(0, STRIPE // B, zs, 0)
    rem = STRIPE - (STRIPE // B) * B
    pltpu.sync_copy(buf.at[pl.ds(0, rem)],
                    acc.at[pl.ds(s * STRIPE + (STRIPE // B) * B, rem)])


def _relation_round(table, gidx, q, sidx, out, ch, c, s,
                    g_i, s_i, rows, gsem, ssem, ig, isx, acc):
    nch = BPT // ch
    _zero_acc(acc, rows[0], s)
    plsc.subcore_barrier()
    pltpu.async_copy(gidx.at[q, pl.ds(s * BPT, ch)], g_i.at[0], ig)
    pltpu.async_copy(sidx.at[pl.ds(s * BPT, ch)], s_i.at[0], isx)

    def chunk(k, carry):
        cur = lax.rem(k, 2)
        nxt = 1 - cur
        pltpu.make_async_copy(gidx.at[q, pl.ds(0, ch)],
                              g_i.at[cur], ig).wait()
        pltpu.make_async_copy(sidx.at[pl.ds(0, ch)], s_i.at[cur], isx).wait()

        @pl.when(k + 1 < nch)
        def _():
            off = s * BPT + (k + 1) * ch
            pltpu.async_copy(gidx.at[q, pl.ds(off, ch)], g_i.at[nxt], ig)
            pltpu.async_copy(sidx.at[pl.ds(off, ch)], s_i.at[nxt], isx)

        gd = [None] * ch
        sd = [None] * ch

        def scat(p):
            gd[p].wait()
            sd[p] = pltpu.async_copy(rows[p % NB], acc.at[s_i.at[cur, p]],
                                     ssem[p % NB], add=True)

        for p in range(ch):
            if p >= NB:
                sd[p - NB].wait()
            gd[p] = pltpu.async_copy(table.at[g_i.at[cur, p]],
                                     rows[p % NB], gsem[p % NB])
            if p >= 2:
                scat(p - 2)
        for p in range(ch - 2, ch):
            scat(p)
        for p in range(ch - NB, ch):
            sd[p].wait()
        return carry

    lax.fori_loop(0, nch, chunk, 0)
    plsc.subcore_barrier()
    pltpu.sync_copy(acc.at[pl.ds(s * STRIPE, STRIPE)],
                    out.at[q, pl.ds(s * STRIPE, STRIPE)])


def _hist(cidx, out, ch, c, s, s_i, ones, ssem, isx, acc):
    nch = BPT // ch
    _zero_acc(acc, ones, s)
    _fill(ones, B, QW, 1.0)
    plsc.subcore_barrier()
    pltpu.async_copy(cidx.at[c, pl.ds(s * BPT, ch)], s_i.at[0], isx)

    def chunk(k, carry):
        cur = lax.rem(k, 2)
        nxt = 1 - cur
        pltpu.make_async_copy(cidx.at[c, pl.ds(0, ch)],
                              s_i.at[cur], isx).wait()

        @pl.when(k + 1 < nch)
        def _():
            off = s * BPT + (k + 1) * ch
            pltpu.async_copy(cidx.at[c, pl.ds(off, ch)], s_i.at[nxt], isx)

        sd = [None] * ch
        for p in range(ch):
            if p >= NB:
                sd[p - NB].wait()
            sd[p] = pltpu.async_copy(ones, acc.at[s_i.at[cur, p]],
                                     ssem[p % NB], add=True)
        for p in range(ch - NB, ch):
            sd[p].wait()
        return carry

    lax.fori_loop(0, nch, chunk, 0)
    plsc.subcore_barrier()
    pltpu.sync_copy(acc.at[pl.ds(s * STRIPE, STRIPE)],
                    out.at[c, pl.ds(s * STRIPE, STRIPE)])


CH_A = 16
CH_B = 8


def _layer1_body(tab_c, tab_v, g1, g2, s1, s2, cidx, out_v, out_c, out_n,
                 g_i, s_i, r0, r1, r2, r3,
                 gs0, gs1, gs2, gs3, ss0, ss1, ss2, ss3, ig, isx, acc):
    c = lax.axis_index("c")
    s = lax.axis_index("s")
    rows = [r0, r1, r2, r3]
    gsem = [gs0, gs1, gs2, gs3]
    ssem = [ss0, ss1, ss2, ss3]
    for r in range(2):
        _relation_round(tab_c, g1, 2 * c + r, s1, out_v, CH_A, c, s,
                        g_i, s_i, rows, gsem, ssem, ig, isx, acc)
    for r in range(2):
        _relation_round(tab_v, g2, 2 * c + r, s2, out_c, CH_A, c, s,
                        g_i, s_i, rows, gsem, ssem, ig, isx, acc)
    _hist(cidx, out_n, CH_A, c, s, s_i, r0, ssem, isx, acc)


def _layer2_body(tab_c, g1, s1, out_v,
                 g_i, s_i, r0, r1, r2, r3,
                 gs0, gs1, gs2, gs3, ss0, ss1, ss2, ss3, ig, isx, acc):
    c = lax.axis_index("c")
    s = lax.axis_index("s")
    rows = [r0, r1, r2, r3]
    gsem = [gs0, gs1, gs2, gs3]
    ssem = [ss0, ss1, ss2, ss3]
    for r in range(2):
        _relation_round(tab_c, g1, 2 * c + r, s1, out_v, CH_B, c, s,
                        g_i, s_i, rows, gsem, ssem, ig, isx, acc)


def _sc_scratch(ch):
    return [
        pltpu.VMEM((2, ch, B), jnp.int32),
        pltpu.VMEM((2, ch, B), jnp.int32),
        pltpu.VMEM((B, QW), jnp.float32),
        pltpu.VMEM((B, QW), jnp.float32),
        pltpu.VMEM((B, QW), jnp.float32),
        pltpu.VMEM((B, QW), jnp.float32),
    ] + [pltpu.SemaphoreType.DMA] * 10 + [
        pltpu.VMEM_SHARED((ACC_R, QW), jnp.float32),
    ]


@functools.cache
def _get_layer1():
    mesh = plsc.VectorSubcoreMesh(core_axis_name="c", subcore_axis_name="s",
                                  num_cores=NC, num_subcores=NS)
    sum_ty = jax.ShapeDtypeStruct((NQ, ACC_R, QW), jnp.float32)
    cnt_ty = jax.ShapeDtypeStruct((NC, ACC_R, QW), jnp.float32)
    return functools.partial(
        pl.kernel,
        out_type=[sum_ty, sum_ty, cnt_ty],
        mesh=mesh,
        scratch_types=_sc_scratch(CH_A),
        compiler_params=pltpu.CompilerParams(use_tc_tiling_on_sc=False),
    )(_layer1_body)


@functools.cache
def _get_layer2():
    mesh = plsc.VectorSubcoreMesh(core_axis_name="c", subcore_axis_name="s",
                                  num_cores=NC, num_subcores=NS)
    sum_ty = jax.ShapeDtypeStruct((NQ, ACC_R, QW), jnp.float32)
    return functools.partial(
        pl.kernel,
        out_type=sum_ty,
        mesh=mesh,
        scratch_types=_sc_scratch(CH_B),
        compiler_params=pltpu.CompilerParams(use_tc_tiling_on_sc=False),
    )(_layer2_body)


def _sc_layer1(*args):
    return _get_layer1()(*args)


def _sc_layer2(*args):
    return _get_layer2()(*args)


RB = 2000
GRID = N_NODE // RB

def _bcast(i):
    return (0, 0)


def _embed_body(cx, vx, bi, cw1, cb1, cw2, cb2, vw1, vb1, vw2, vb2, bw,
                ch_o, vh_o):
    f32 = jnp.float32
    ch = jnp.maximum(jnp.dot(cx[...], cw1[...], preferred_element_type=f32)
                     + cb1[...], 0.0)
    ch = jnp.maximum(jnp.dot(ch, cw2[...], preferred_element_type=f32)
                     + cb2[...], 0.0)
    vh = jnp.maximum(jnp.dot(vx[...], vw1[...], preferred_element_type=f32)
                     + vb1[...], 0.0)
    vh = jnp.maximum(jnp.dot(vh, vw2[...], preferred_element_type=f32)
                     + vb2[...], 0.0)
    vh = vh + bi[...] * bw[...]
    ch_o[...] = ch
    vh_o[...] = vh


def _make_embed():
    wspec = lambda shp: pl.BlockSpec(shp, _bcast)
    return pl.pallas_call(
        _embed_body,
        grid=(GRID,),
        in_specs=[
            pl.BlockSpec((RB, 8), lambda i: (i, 0)),
            pl.BlockSpec((RB, 24), lambda i: (i, 0)),
            pl.BlockSpec((RB, 1), lambda i: (i, 0)),
            wspec((8, EMB)), wspec((1, EMB)),
            wspec((EMB, EMB)), wspec((1, EMB)),
            wspec((24, EMB)), wspec((1, EMB)),
            wspec((EMB, EMB)), wspec((1, EMB)),
            wspec((1, EMB)),
        ],
        out_specs=[
            pl.BlockSpec((RB, EMB), lambda i: (i, 0)),
            pl.BlockSpec((RB, EMB), lambda i: (i, 0)),
        ],
        out_shape=[
            jax.ShapeDtypeStruct((N_NODE, EMB), jnp.float32),
            jax.ShapeDtypeStruct((N_NODE, EMB), jnp.float32),
        ],
    )


def _upd_side_body(sm, cnts, h_r, wl, bl, wr, h_o):
    f32 = jnp.float32
    sm_a = sm[...]
    cn_a = cnts[...]
    mean = (jnp.concatenate([sm_a[q] for q in range(NQ)], axis=1)
            / jnp.maximum(cn_a[0][:, 0:1], 1.0))
    nh = (jnp.dot(mean, wl[...], preferred_element_type=f32) + bl[...]
          + jnp.dot(h_r[...], wr[...], preferred_element_type=f32))
    h_o[...] = jnp.maximum(nh, 0.0)


def _make_upd_side(cslot):
    wspec = lambda shp: pl.BlockSpec(shp, _bcast)
    return pl.pallas_call(
        _upd_side_body,
        grid=(GRID,),
        in_specs=[
            pl.BlockSpec((NQ, RB, QW), lambda i: (0, i, 0)),
            pl.BlockSpec((1, RB, QW), lambda i: (cslot, i, 0)),
            pl.BlockSpec((RB, EMB), lambda i: (i, 0)),
            wspec((EMB, EMB)), wspec((1, EMB)), wspec((EMB, EMB)),
        ],
        out_specs=pl.BlockSpec((RB, EMB), lambda i: (i, 0)),
        out_shape=jax.ShapeDtypeStruct((N_NODE, EMB), jnp.float32),
    )


_embed_call = _make_embed()
_upd_var_call = _make_upd_side(0)
_upd_cons_call = _make_upd_side(1)


def kernel(cons_x, var_x, edge_index, edge_attr, break_indicator,
           cons_shift, cons_scale, cons_W1, cons_b1, cons_W2, cons_b2,
           var_shift, var_scale, var_W1, var_b1, var_W2, var_b2,
           edge_shift, edge_scale, break_W, lin_l_W, lin_l_b, lin_r_W):
    del edge_attr, edge_shift, edge_scale

    cw1 = cons_scale[:, None] * cons_W1
    cb1 = (cons_b1 + (cons_shift * cons_scale) @ cons_W1)[None, :]
    vw1 = var_scale[:, None] * var_W1
    vb1 = (var_b1 + (var_shift * var_scale) @ var_W1)[None, :]
    cx = jnp.pad(cons_x, ((0, 0), (0, 3)))
    vx = jnp.pad(var_x, ((0, 0), (0, 5)))
    cw1 = jnp.pad(cw1, ((0, 3), (0, 0)))
    vw1 = jnp.pad(vw1, ((0, 5), (0, 0)))

    src = edge_index[0].astype(jnp.int32)
    dst = edge_index[1].astype(jnp.int32)
    padn = EP - N_EDGE
    src_g = jnp.pad(src, (0, padn))
    dst_g = jnp.pad(dst, (0, padn))
    src_s = jnp.pad(src, (0, padn), constant_values=N_NODE)
    dst_s = jnp.pad(dst, (0, padn), constant_values=N_NODE)

    def gidx_of(x):
        return jnp.stack([NQ * x + q for q in range(NQ)]).reshape(
            NQ, NS * BPT, B)

    g_rel1 = gidx_of(src_g)
    g_rel2 = gidx_of(dst_g)
    s_rel1 = dst_s.reshape(NS * BPT, B)
    s_rel2 = src_s.reshape(NS * BPT, B)
    c_idx = jnp.stack([dst_s, src_s]).reshape(NC, NS * BPT, B)

    ch0, vh0 = _embed_call(cx, vx, break_indicator, cw1, cb1,
                           cons_W2, cons_b2[None, :], vw1, vb1,
                           var_W2, var_b2[None, :], break_W)

    sv1, sc1, cn = _sc_layer1(ch0.reshape(NQ * N_NODE, QW),
                              vh0.reshape(NQ * N_NODE, QW),
                              g_rel1, g_rel2, s_rel1, s_rel2, c_idx)
    ch1 = _upd_cons_call(sc1, cn, ch0,
                         lin_l_W[0, 1], lin_l_b[0, 1][None, :],
                         lin_r_W[0, 1])

    sv2 = _sc_layer2(ch1.reshape(NQ * N_NODE, QW), g_rel1, s_rel1)
    vh1 = _upd_var_call(sv1, cn, vh0,
                        lin_l_W[0, 0], lin_l_b[0, 0][None, :],
                        lin_r_W[0, 0])
    vh2 = _upd_var_call(sv2, cn, vh1,
                        lin_l_W[1, 0], lin_l_b[1, 0][None, :],
                        lin_r_W[1, 0])
    return vh2
